# Initial kernel scaffold; baseline (speedup 1.0000x reference)
import jax
import jax.numpy as jnp
from jax.experimental import pallas as pl


def kernel(x, edge_index, phi, sim, params):
    raise NotImplementedError("write your pallas kernel here")


# speedup vs baseline: 9.8441x; 9.8441x over previous
import functools

import jax
import jax.numpy as jnp
import numpy as np
from jax.experimental import pallas as pl
from jax.experimental.pallas import tpu as pltpu

H = 8
DH = 16
HID = 128

_S16 = np.kron(np.eye(H, dtype=np.float32), np.ones((DH, 1), np.float32))
_R8 = np.kron(np.eye(H, dtype=np.float32), np.ones((1, DH), np.float32))

BN = 1000
BE = 1600


def _node_tables_body(h_ref, ws_ref, bs_ref, wd_ref, bd_ref, s_ref, d_ref):
    h = h_ref[...]
    s_ref[...] = h @ ws_ref[...] + bs_ref[...]
    d_ref[...] = h @ wd_ref[...] + bd_ref[...]


def _node_tables(h, WS, bS, WD, bD):
    n = h.shape[0]
    grid = (n // BN,)
    return pl.pallas_call(
        _node_tables_body,
        grid=grid,
        in_specs=[
            pl.BlockSpec((BN, HID), lambda i: (i, 0)),
            pl.BlockSpec(WS.shape, lambda i: (0, 0)),
            pl.BlockSpec(bS.shape, lambda i: (0, 0)),
            pl.BlockSpec(WD.shape, lambda i: (0, 0)),
            pl.BlockSpec(bD.shape, lambda i: (0, 0)),
        ],
        out_specs=[
            pl.BlockSpec((BN, WS.shape[1]), lambda i: (i, 0)),
            pl.BlockSpec((BN, WD.shape[1]), lambda i: (i, 0)),
        ],
        out_shape=[
            jax.ShapeDtypeStruct((n, WS.shape[1]), jnp.float32),
            jax.ShapeDtypeStruct((n, WD.shape[1]), jnp.float32),
        ],
    )(h, WS, bS, WD, bD)


def _edge_const_body(sim_ref, phi_ref, wc_ref, out_ref):
    sim = sim_ref[...]
    wc = wc_ref[...]
    acc = jnp.zeros_like(out_ref[...]) + wc[0][None, :]
    acc = acc + phi_ref[...] * wc[1][None, :]
    feats = []
    for i in range(4):
        feats.append(sim[:, i:i + 1])
    k = 2
    for i in range(4):
        acc = acc + feats[i] * wc[k + i][None, :]
    k = 6
    for i in range(4):
        for j in range(i, 4):
            acc = acc + (feats[i] * feats[j]) * wc[k][None, :]
            k += 1
    out_ref[...] = acc


def _edge_const(sim, phi, Wc):
    e = sim.shape[0]
    return pl.pallas_call(
        _edge_const_body,
        grid=(e // BE,),
        in_specs=[
            pl.BlockSpec((BE, 4), lambda i: (i, 0)),
            pl.BlockSpec((BE, 1), lambda i: (i, 0)),
            pl.BlockSpec(Wc.shape, lambda i: (0, 0)),
        ],
        out_specs=pl.BlockSpec((BE, Wc.shape[1]), lambda i: (i, 0)),
        out_shape=jax.ShapeDtypeStruct((e, Wc.shape[1]), jnp.float32),
    )(sim, phi, Wc)


def _edge_score_body(gs_ref, gd_ref, sim_ref, c_ref, s16_ref, r8_ref, y_ref):
    gs = gs_ref[...]
    gd = gd_ref[...]
    sim = sim_ref[...]
    ks = gs[:, :HID]
    vs = gs[:, HID:2 * HID]
    aK = gs[:, 2 * HID:2 * HID + 48]
    qd = gd[:, :HID]
    aQ = gd[:, HID:HID + 48]
    dots = (ks * qd) @ s16_ref[...]
    a = aK + aQ
    term = a[:, 32:40]
    for j in range(4):
        term = term + sim[:, j:j + 1] * a[:, j * 8:j * 8 + 8]
    s = dots + term + c_ref[...]
    w = jnp.exp(jnp.clip(s, -8.0, 8.0))
    wex = w @ r8_ref[...]
    y_ref[:, :HID] = vs * wex
    y_ref[:, HID:HID + 16] = jnp.pad(w, ((0, 0), (0, 8)))


def _edge_scores(Gs, Gd, sim, C):
    e = sim.shape[0]
    return pl.pallas_call(
        _edge_score_body,
        grid=(e // BE,),
        in_specs=[
            pl.BlockSpec((BE, Gs.shape[1]), lambda i: (i, 0)),
            pl.BlockSpec((BE, Gd.shape[1]), lambda i: (i, 0)),
            pl.BlockSpec((BE, 4), lambda i: (i, 0)),
            pl.BlockSpec((BE, 8), lambda i: (i, 0)),
            pl.BlockSpec(_S16.shape, lambda i: (0, 0)),
            pl.BlockSpec(_R8.shape, lambda i: (0, 0)),
        ],
        out_specs=pl.BlockSpec((BE, 144), lambda i: (i, 0)),
        out_shape=jax.ShapeDtypeStruct((e, 144), jnp.float32),
    )(Gs, Gd, sim, C, jnp.asarray(_S16), jnp.asarray(_R8))


def _ln(x, g, b):
    m = jnp.mean(x, axis=-1, keepdims=True)
    xc = x - m
    v = jnp.mean(xc * xc, axis=-1, keepdims=True)
    return xc * jax.lax.rsqrt(v + 1e-5) * g + b


def _post_body(p_ref, h_ref, wo_ref, bo_ref, g1_ref, b1g_ref, w1_ref, b1_ref,
               w2_ref, b2_ref, g2_ref, b2g_ref, r8_ref, o_ref):
    p = p_ref[...]
    wvz = jnp.sum(p, axis=0)
    wV = wvz[:, :HID]
    z = wvz[:, HID:HID + 8]
    attn = wV / (z @ r8_ref[...] + 1e-6)
    hh = attn @ wo_ref[...] + bo_ref[...]
    r1 = _ln(h_ref[...] + hh, g1_ref[...], b1g_ref[...])
    f = jnp.maximum(r1 @ w1_ref[...] + b1_ref[...], 0.0) @ w2_ref[...] + b2_ref[...]
    o_ref[...] = _ln(r1 + f, g2_ref[...], b2g_ref[...])


def _post(parts, h, Wo, bo, g1, b1g, W1, b1, W2, b2, g2, b2g):
    n = h.shape[0]
    P = parts.shape[0]
    full = lambda a: pl.BlockSpec(a.shape, lambda i: (0,) * a.ndim)
    return pl.pallas_call(
        _post_body,
        grid=(n // BN,),
        in_specs=[
            pl.BlockSpec((P, BN, 144), lambda i: (0, i, 0)),
            pl.BlockSpec((BN, HID), lambda i: (i, 0)),
            full(Wo), full(bo), full(g1), full(b1g), full(W1), full(b1),
            full(W2), full(b2), full(g2), full(b2g),
            pl.BlockSpec(_R8.shape, lambda i: (0, 0)),
        ],
        out_specs=pl.BlockSpec((BN, HID), lambda i: (i, 0)),
        out_shape=jax.ShapeDtypeStruct((n, HID), jnp.float32),
    )(parts, h, Wo, bo, g1, b1g, W1, b1, W2, b2, g2, b2g, jnp.asarray(_R8))


def _embed_body(x_ref, w_ref, o_ref):
    o_ref[...] = x_ref[...] @ w_ref[...]


def _embed(x, W):
    n = x.shape[0]
    return pl.pallas_call(
        _embed_body,
        grid=(n // BN,),
        in_specs=[
            pl.BlockSpec((BN, x.shape[1]), lambda i: (i, 0)),
            pl.BlockSpec(W.shape, lambda i: (0, 0)),
        ],
        out_specs=pl.BlockSpec((BN, HID), lambda i: (i, 0)),
        out_shape=jax.ShapeDtypeStruct((n, HID), jnp.float32),
    )(x, W)


def _prep_layer(params, p):
    Ms = jnp.concatenate(
        [params['emb_sim_W'] @ p['Wsim'],
         (params['emb_sim_b'] @ p['Wsim'] + p['bsim'])[None, :]], axis=0)
    M5 = Ms.reshape(5, H, DH)
    kr = jnp.asarray(np.kron(np.eye(H, dtype=np.float32), np.ones((DH, 1), np.float32)))
    T = jnp.concatenate([4.0 * Ms[j][:, None] * kr for j in range(5)]
                        + [jnp.zeros((HID, 8), jnp.float32)], axis=1)
    WS = jnp.concatenate([2.0 * p['Wk'], p['Wv'], p['Wk'] @ T], axis=1)
    bS = jnp.concatenate([2.0 * p['bk'], p['bv'], p['bk'] @ T])[None, :]
    WD = jnp.concatenate([2.0 * p['Wq'], p['Wq'] @ T], axis=1)
    bD = jnp.concatenate([2.0 * p['bq'], p['bq'] @ T])[None, :]

    Mp = (params['emb_phi_W'] @ p['Wphi'])[0]
    cp = params['emb_phi_b'] @ p['Wphi'] + p['bphi']
    Psum = Mp.reshape(H, DH).sum(-1)
    Csum = cp.reshape(H, DH).sum(-1)
    G = jnp.einsum('jhd,khd->hjk', M5, M5)
    rows = [4.0 * G[:, 4, 4] + Csum, Psum]
    for i in range(4):
        rows.append(8.0 * G[:, i, 4])
    for i in range(4):
        for j in range(i, 4):
            rows.append((4.0 if i == j else 8.0) * G[:, i, j])
    Wc = jnp.stack(rows, axis=0)
    return WS, bS, WD, bD, Wc


def kernel(x, edge_index, phi, sim, params):
    src = edge_index[0]
    dst = edge_index[1]
    n = x.shape[0]
    e = sim.shape[0]

    layer_prep = [_prep_layer(params, p) for p in params['layers']]
    Wc_all = jnp.concatenate([lp[4] for lp in layer_prep], axis=1)

    h = _embed(x, params['emb_h_W'])
    C_all = _edge_const(sim, phi, Wc_all)

    for li, p in enumerate(params['layers']):
        WS, bS, WD, bD, _ = layer_prep[li]
        nodeS, nodeD = _node_tables(h, WS, bS, WD, bD)
        Gs = jnp.take(nodeS, src, axis=0)
        Gd = jnp.take(nodeD, dst, axis=0)
        Y = _edge_scores(Gs, Gd, sim, C_all[:, li * 8:(li + 1) * 8])
        seg = jax.ops.segment_sum(Y, dst, num_segments=n)
        h = _post(seg[None], h, p['Wo'], p['bo'],
                  p['ln1_g'][None, :], p['ln1_b'][None, :],
                  p['W1'], p['b1'][None, :], p['W2'], p['b2'][None, :],
                  p['ln2_g'][None, :], p['ln2_b'][None, :])
    return h


# speedup vs baseline: 19.5462x; 1.9856x over previous
import functools

import jax
import jax.numpy as jnp
import numpy as np
from jax import lax
from jax.experimental import pallas as pl
from jax.experimental.pallas import tpu as pltpu
from jax.experimental.pallas import tpu_sc as plsc

H = 8
DH = 16
HID = 128

_S16 = np.kron(np.eye(H, dtype=np.float32), np.ones((DH, 1), np.float32))
_R8 = np.kron(np.eye(H, dtype=np.float32), np.ones((1, DH), np.float32))

BN = 1000
BE = 1600


def _node_tables_body(h_ref, ws_ref, bs_ref, wd_ref, bd_ref, s_ref, d_ref):
    h = h_ref[...]
    s_ref[...] = h @ ws_ref[...] + bs_ref[...]
    d_ref[...] = h @ wd_ref[...] + bd_ref[...]


def _node_tables(h, WS, bS, WD, bD):
    n = h.shape[0]
    grid = (n // BN,)
    return pl.pallas_call(
        _node_tables_body,
        grid=grid,
        in_specs=[
            pl.BlockSpec((BN, HID), lambda i: (i, 0)),
            pl.BlockSpec(WS.shape, lambda i: (0, 0)),
            pl.BlockSpec(bS.shape, lambda i: (0, 0)),
            pl.BlockSpec(WD.shape, lambda i: (0, 0)),
            pl.BlockSpec(bD.shape, lambda i: (0, 0)),
        ],
        out_specs=[
            pl.BlockSpec((BN, WS.shape[1]), lambda i: (i, 0)),
            pl.BlockSpec((BN, WD.shape[1]), lambda i: (i, 0)),
        ],
        out_shape=[
            jax.ShapeDtypeStruct((n, WS.shape[1]), jnp.float32),
            jax.ShapeDtypeStruct((n, WD.shape[1]), jnp.float32),
        ],
    )(h, WS, bS, WD, bD)


def _edge_const_body(sim_ref, phi_ref, wc_ref, out_ref):
    sim = sim_ref[...]
    wc = wc_ref[...]
    acc = jnp.zeros_like(out_ref[...]) + wc[0][None, :]
    acc = acc + phi_ref[...] * wc[1][None, :]
    feats = []
    for i in range(4):
        feats.append(sim[:, i:i + 1])
    k = 2
    for i in range(4):
        acc = acc + feats[i] * wc[k + i][None, :]
    k = 6
    for i in range(4):
        for j in range(i, 4):
            acc = acc + (feats[i] * feats[j]) * wc[k][None, :]
            k += 1
    out_ref[...] = acc


def _edge_const(sim, phi, Wc):
    e = sim.shape[0]
    return pl.pallas_call(
        _edge_const_body,
        grid=(e // BE,),
        in_specs=[
            pl.BlockSpec((BE, 4), lambda i: (i, 0)),
            pl.BlockSpec((BE, 1), lambda i: (i, 0)),
            pl.BlockSpec(Wc.shape, lambda i: (0, 0)),
        ],
        out_specs=pl.BlockSpec((BE, Wc.shape[1]), lambda i: (i, 0)),
        out_shape=jax.ShapeDtypeStruct((e, Wc.shape[1]), jnp.float32),
    )(sim, phi, Wc)


def _edge_score_body(gs_ref, gd_ref, sim_ref, c_ref, s16_ref, r8_ref, y_ref):
    gs = gs_ref[...]
    gd = gd_ref[...]
    sim = sim_ref[...]
    ks = gs[:, :HID]
    vs = gs[:, HID:2 * HID]
    aK = gs[:, 2 * HID:2 * HID + 48]
    qd = gd[:, :HID]
    aQ = gd[:, HID:HID + 48]
    dots = (ks * qd) @ s16_ref[...]
    a = aK + aQ
    term = a[:, 32:40]
    for j in range(4):
        term = term + sim[:, j:j + 1] * a[:, j * 8:j * 8 + 8]
    s = dots + term + c_ref[...]
    w = jnp.exp(jnp.clip(s, -8.0, 8.0))
    wex = w @ r8_ref[...]
    y_ref[:, :HID] = vs * wex
    y_ref[:, HID:2 * HID] = jnp.pad(w, ((0, 0), (0, HID - 8)))


def _edge_scores(Gs, Gd, sim, C):
    e = sim.shape[0]
    return pl.pallas_call(
        _edge_score_body,
        grid=(e // BE,),
        in_specs=[
            pl.BlockSpec((BE, Gs.shape[1]), lambda i: (i, 0)),
            pl.BlockSpec((BE, Gd.shape[1]), lambda i: (i, 0)),
            pl.BlockSpec((BE, 4), lambda i: (i, 0)),
            pl.BlockSpec((BE, 8), lambda i: (i, 0)),
            pl.BlockSpec(_S16.shape, lambda i: (0, 0)),
            pl.BlockSpec(_R8.shape, lambda i: (0, 0)),
        ],
        out_specs=pl.BlockSpec((BE, 256), lambda i: (i, 0)),
        out_shape=jax.ShapeDtypeStruct((e, 256), jnp.float32),
    )(Gs, Gd, sim, C, jnp.asarray(_S16), jnp.asarray(_R8))


def _ln(x, g, b):
    m = jnp.mean(x, axis=-1, keepdims=True)
    xc = x - m
    v = jnp.mean(xc * xc, axis=-1, keepdims=True)
    return xc * jax.lax.rsqrt(v + 1e-5) * g + b


def _post_body(p_ref, h_ref, wo_ref, bo_ref, g1_ref, b1g_ref, w1_ref, b1_ref,
               w2_ref, b2_ref, g2_ref, b2g_ref, r8_ref, o_ref):
    p = p_ref[...]
    wV = p[0]
    z = p[1][:, :8]
    attn = wV / (z @ r8_ref[...] + 1e-6)
    hh = attn @ wo_ref[...] + bo_ref[...]
    r1 = _ln(h_ref[...] + hh, g1_ref[...], b1g_ref[...])
    f = jnp.maximum(r1 @ w1_ref[...] + b1_ref[...], 0.0) @ w2_ref[...] + b2_ref[...]
    o_ref[...] = _ln(r1 + f, g2_ref[...], b2g_ref[...])


def _post(parts, h, Wo, bo, g1, b1g, W1, b1, W2, b2, g2, b2g):
    n = h.shape[0]
    P = parts.shape[0]
    full = lambda a: pl.BlockSpec(a.shape, lambda i: (0,) * a.ndim)
    return pl.pallas_call(
        _post_body,
        grid=(n // BN,),
        in_specs=[
            pl.BlockSpec((P, BN, HID), lambda i: (0, i, 0)),
            pl.BlockSpec((BN, HID), lambda i: (i, 0)),
            full(Wo), full(bo), full(g1), full(b1g), full(W1), full(b1),
            full(W2), full(b2), full(g2), full(b2g),
            pl.BlockSpec(_R8.shape, lambda i: (0, 0)),
        ],
        out_specs=pl.BlockSpec((BN, HID), lambda i: (i, 0)),
        out_shape=jax.ShapeDtypeStruct((n, HID), jnp.float32),
    )(parts, h, Wo, bo, g1, b1g, W1, b1, W2, b2, g2, b2g, jnp.asarray(_R8))


def _embed_body(x_ref, w_ref, o_ref):
    o_ref[...] = x_ref[...] @ w_ref[...]


def _embed(x, W):
    n = x.shape[0]
    return pl.pallas_call(
        _embed_body,
        grid=(n // BN,),
        in_specs=[
            pl.BlockSpec((BN, x.shape[1]), lambda i: (i, 0)),
            pl.BlockSpec(W.shape, lambda i: (0, 0)),
        ],
        out_specs=pl.BlockSpec((BN, HID), lambda i: (i, 0)),
        out_shape=jax.ShapeDtypeStruct((n, HID), jnp.float32),
    )(x, W)


_NCORES = 2
_NSUB = 16
_NW = _NCORES * _NSUB
_GCH = 80


def _sc_gather(nodeS, nodeD, src, dst):
    e = src.shape[0]
    ws, wd = nodeS.shape[1], nodeD.shape[1]
    per_w = e // _NW
    nch = per_w // _GCH
    mesh = plsc.VectorSubcoreMesh(core_axis_name="c", subcore_axis_name="s")

    @functools.partial(
        pl.kernel, mesh=mesh,
        out_type=[jax.ShapeDtypeStruct((e, ws), jnp.float32),
                  jax.ShapeDtypeStruct((e, wd), jnp.float32)],
        scratch_types=[pltpu.VMEM((_GCH,), jnp.int32),
                       pltpu.VMEM((_GCH,), jnp.int32),
                       pltpu.VMEM((_GCH, ws), jnp.float32),
                       pltpu.VMEM((_GCH, wd), jnp.float32),
                       pltpu.SemaphoreType.DMA,
                       pltpu.SemaphoreType.DMA],
    )
    def k(ns_hbm, nd_hbm, src_hbm, dst_hbm, gs_hbm, gd_hbm,
          si_v, di_v, rs_v, rd_v, sem1, sem2):
        wid = lax.axis_index("s") * _NCORES + lax.axis_index("c")
        base = wid * per_w

        def body(ci, carry):
            off = base + ci * _GCH
            pltpu.sync_copy(src_hbm.at[pl.ds(off, _GCH)], si_v)
            pltpu.sync_copy(dst_hbm.at[pl.ds(off, _GCH)], di_v)
            cp1 = pltpu.async_copy(ns_hbm.at[si_v], rs_v, sem1)
            cp2 = pltpu.async_copy(nd_hbm.at[di_v], rd_v, sem2)
            cp1.wait()
            cp2.wait()
            pltpu.sync_copy(rs_v, gs_hbm.at[pl.ds(off, _GCH)])
            pltpu.sync_copy(rd_v, gd_hbm.at[pl.ds(off, _GCH)])
            return carry

        lax.fori_loop(0, nch, body, 0)

    return k(nodeS, nodeD, src, dst)


def _sc_scatter_add(Y, dst, n):
    e, wy = Y.shape
    per_tile = e // _NSUB
    nch = per_tile // _GCH
    npad = ((n + 8 * _NSUB - 1) // (8 * _NSUB)) * (8 * _NSUB)
    rows_pt = npad // _NSUB
    mesh = plsc.VectorSubcoreMesh(core_axis_name="c", subcore_axis_name="s")
    zeros_hbm_in = jnp.zeros((npad, HID), jnp.float32)

    @functools.partial(
        pl.kernel, mesh=mesh,
        out_type=jax.ShapeDtypeStruct((_NCORES, npad, HID), jnp.float32),
        scratch_types=[pltpu.VMEM((_GCH,), jnp.int32),
                       pltpu.VMEM((_GCH, HID), jnp.float32),
                       pltpu.VMEM_SHARED((npad, HID), jnp.float32),
                       pltpu.SemaphoreType.DMA],
    )
    def k(y_hbm, dst_hbm, zz_hbm, out_hbm, di_v, y_v, acc_sh, sem):
        cid = lax.axis_index("c")
        sid = lax.axis_index("s")

        pltpu.sync_copy(zz_hbm.at[pl.ds(sid * rows_pt, rows_pt)],
                        acc_sh.at[pl.ds(sid * rows_pt, rows_pt)])
        plsc.subcore_barrier()

        base = sid * per_tile
        col = cid * HID

        def body(ci, carry):
            off = base + ci * _GCH
            pltpu.sync_copy(dst_hbm.at[pl.ds(off, _GCH)], di_v)
            pltpu.async_copy(y_hbm.at[pl.ds(off, _GCH), pl.ds(col, HID)],
                             y_v, sem).wait()
            pltpu.sync_copy(y_v, acc_sh.at[di_v], add=True)
            return carry
        lax.fori_loop(0, nch, body, 0)
        plsc.subcore_barrier()

        pltpu.sync_copy(acc_sh.at[pl.ds(sid * rows_pt, rows_pt)],
                        out_hbm.at[cid, pl.ds(sid * rows_pt, rows_pt)])

    return k(Y, dst, zeros_hbm_in)


def _prep_layer(params, p):
    Ms = jnp.concatenate(
        [params['emb_sim_W'] @ p['Wsim'],
         (params['emb_sim_b'] @ p['Wsim'] + p['bsim'])[None, :]], axis=0)
    M5 = Ms.reshape(5, H, DH)
    kr = jnp.asarray(np.kron(np.eye(H, dtype=np.float32), np.ones((DH, 1), np.float32)))
    T = jnp.concatenate([4.0 * Ms[j][:, None] * kr for j in range(5)]
                        + [jnp.zeros((HID, 88), jnp.float32)], axis=1)
    WS = jnp.concatenate([2.0 * p['Wk'], p['Wv'], p['Wk'] @ T], axis=1)
    bS = jnp.concatenate([2.0 * p['bk'], p['bv'], p['bk'] @ T])[None, :]
    WD = jnp.concatenate([2.0 * p['Wq'], p['Wq'] @ T], axis=1)
    bD = jnp.concatenate([2.0 * p['bq'], p['bq'] @ T])[None, :]

    Mp = (params['emb_phi_W'] @ p['Wphi'])[0]
    cp = params['emb_phi_b'] @ p['Wphi'] + p['bphi']
    Psum = Mp.reshape(H, DH).sum(-1)
    Csum = cp.reshape(H, DH).sum(-1)
    G = jnp.einsum('jhd,khd->hjk', M5, M5)
    rows = [4.0 * G[:, 4, 4] + Csum, Psum]
    for i in range(4):
        rows.append(8.0 * G[:, i, 4])
    for i in range(4):
        for j in range(i, 4):
            rows.append((4.0 if i == j else 8.0) * G[:, i, j])
    Wc = jnp.stack(rows, axis=0)
    return WS, bS, WD, bD, Wc


def kernel(x, edge_index, phi, sim, params):
    src = edge_index[0]
    dst = edge_index[1]
    n = x.shape[0]
    e = sim.shape[0]

    layer_prep = [_prep_layer(params, p) for p in params['layers']]
    Wc_all = jnp.concatenate([lp[4] for lp in layer_prep], axis=1)

    h = _embed(x, params['emb_h_W'])
    C_all = _edge_const(sim, phi, Wc_all)

    for li, p in enumerate(params['layers']):
        WS, bS, WD, bD, _ = layer_prep[li]
        nodeS, nodeD = _node_tables(h, WS, bS, WD, bD)
        Gs, Gd = _sc_gather(nodeS, nodeD, src, dst)
        Y = _edge_scores(Gs, Gd, sim, C_all[:, li * 8:(li + 1) * 8])
        seg = _sc_scatter_add(Y, dst, n)
        h = _post(seg, h, p['Wo'], p['bo'],
                  p['ln1_g'][None, :], p['ln1_b'][None, :],
                  p['W1'], p['b1'][None, :], p['W2'], p['b2'][None, :],
                  p['ln2_g'][None, :], p['ln2_b'][None, :])
    return h


# speedup vs baseline: 20.7565x; 1.0619x over previous
import functools

import jax
import jax.numpy as jnp
import numpy as np
from jax import lax
from jax.experimental import pallas as pl
from jax.experimental.pallas import tpu as pltpu
from jax.experimental.pallas import tpu_sc as plsc

H = 8
DH = 16
HID = 128

_S16 = np.kron(np.eye(H, dtype=np.float32), np.ones((DH, 1), np.float32))
_R8 = np.kron(np.eye(H, dtype=np.float32), np.ones((1, DH), np.float32))

BN = 1000
BE = 1600


def _node_tables_body(h_ref, ws_ref, bs_ref, wd_ref, bd_ref, s_ref, d_ref):
    h = h_ref[...]
    s_ref[...] = h @ ws_ref[...] + bs_ref[...]
    d_ref[...] = h @ wd_ref[...] + bd_ref[...]


def _node_tables(h, WS, bS, WD, bD):
    n = h.shape[0]
    grid = (n // BN,)
    return pl.pallas_call(
        _node_tables_body,
        grid=grid,
        in_specs=[
            pl.BlockSpec((BN, HID), lambda i: (i, 0)),
            pl.BlockSpec(WS.shape, lambda i: (0, 0)),
            pl.BlockSpec(bS.shape, lambda i: (0, 0)),
            pl.BlockSpec(WD.shape, lambda i: (0, 0)),
            pl.BlockSpec(bD.shape, lambda i: (0, 0)),
        ],
        out_specs=[
            pl.BlockSpec((BN, WS.shape[1]), lambda i: (i, 0)),
            pl.BlockSpec((BN, WD.shape[1]), lambda i: (i, 0)),
        ],
        out_shape=[
            jax.ShapeDtypeStruct((n, WS.shape[1]), jnp.float32),
            jax.ShapeDtypeStruct((n, WD.shape[1]), jnp.float32),
        ],
    )(h, WS, bS, WD, bD)


def _edge_const_body(sim_ref, phi_ref, wc_ref, out_ref):
    sim = sim_ref[...]
    wc = wc_ref[...]
    acc = jnp.zeros_like(out_ref[...]) + wc[0][None, :]
    acc = acc + phi_ref[...] * wc[1][None, :]
    feats = []
    for i in range(4):
        feats.append(sim[:, i:i + 1])
    k = 2
    for i in range(4):
        acc = acc + feats[i] * wc[k + i][None, :]
    k = 6
    for i in range(4):
        for j in range(i, 4):
            acc = acc + (feats[i] * feats[j]) * wc[k][None, :]
            k += 1
    out_ref[...] = acc


def _edge_const(sim, phi, Wc):
    e = sim.shape[0]
    return pl.pallas_call(
        _edge_const_body,
        grid=(e // BE,),
        in_specs=[
            pl.BlockSpec((BE, 4), lambda i: (i, 0)),
            pl.BlockSpec((BE, 1), lambda i: (i, 0)),
            pl.BlockSpec(Wc.shape, lambda i: (0, 0)),
        ],
        out_specs=pl.BlockSpec((BE, Wc.shape[1]), lambda i: (i, 0)),
        out_shape=jax.ShapeDtypeStruct((e, Wc.shape[1]), jnp.float32),
    )(sim, phi, Wc)


def _edge_score_body(gs_ref, gd_ref, sim_ref, c_ref, s16_ref, r8_ref, y_ref):
    gs = gs_ref[...]
    gd = gd_ref[...]
    sim = sim_ref[...]
    ks = gs[:, :HID]
    vs = gs[:, HID:2 * HID]
    aK = gs[:, 2 * HID:2 * HID + 48]
    qd = gd[:, :HID]
    aQ = gd[:, HID:HID + 48]
    dots = (ks * qd) @ s16_ref[...]
    a = aK + aQ
    term = a[:, 32:40]
    for j in range(4):
        term = term + sim[:, j:j + 1] * a[:, j * 8:j * 8 + 8]
    s = dots + term + c_ref[...]
    w = jnp.exp(jnp.clip(s, -8.0, 8.0))
    wex = w @ r8_ref[...]
    y_ref[:, :HID] = vs * wex
    y_ref[:, HID:2 * HID] = jnp.pad(w, ((0, 0), (0, HID - 8)))


def _edge_scores(Gs, Gd, sim, C):
    e = sim.shape[0]
    return pl.pallas_call(
        _edge_score_body,
        grid=(e // BE,),
        in_specs=[
            pl.BlockSpec((BE, Gs.shape[1]), lambda i: (i, 0)),
            pl.BlockSpec((BE, Gd.shape[1]), lambda i: (i, 0)),
            pl.BlockSpec((BE, 4), lambda i: (i, 0)),
            pl.BlockSpec((BE, 8), lambda i: (i, 0)),
            pl.BlockSpec(_S16.shape, lambda i: (0, 0)),
            pl.BlockSpec(_R8.shape, lambda i: (0, 0)),
        ],
        out_specs=pl.BlockSpec((BE, 256), lambda i: (i, 0)),
        out_shape=jax.ShapeDtypeStruct((e, 256), jnp.float32),
    )(Gs, Gd, sim, C, jnp.asarray(_S16), jnp.asarray(_R8))


def _ln(x, g, b):
    m = jnp.mean(x, axis=-1, keepdims=True)
    xc = x - m
    v = jnp.mean(xc * xc, axis=-1, keepdims=True)
    return xc * jax.lax.rsqrt(v + 1e-5) * g + b


def _make_post_body(nparts):
    def body(*refs):
        p_refs = refs[:nparts]
        (h_ref, wo_ref, bo_ref, g1_ref, b1g_ref, w1_ref, b1_ref,
         w2_ref, b2_ref, g2_ref, b2g_ref, r8_ref, o_ref) = refs[nparts:]
        wV = p_refs[0][0]
        z = p_refs[0][1][:, :8]
        for pr in p_refs[1:]:
            wV = wV + pr[0]
            z = z + pr[1][:, :8]
        attn = wV / (z @ r8_ref[...] + 1e-6)
        hh = attn @ wo_ref[...] + bo_ref[...]
        r1 = _ln(h_ref[...] + hh, g1_ref[...], b1g_ref[...])
        f = jnp.maximum(r1 @ w1_ref[...] + b1_ref[...], 0.0) @ w2_ref[...] + b2_ref[...]
        o_ref[...] = _ln(r1 + f, g2_ref[...], b2g_ref[...])
    return body


def _post(parts_list, h, Wo, bo, g1, b1g, W1, b1, W2, b2, g2, b2g):
    n = h.shape[0]
    full = lambda a: pl.BlockSpec(a.shape, lambda i: (0,) * a.ndim)
    return pl.pallas_call(
        _make_post_body(len(parts_list)),
        grid=(n // BN,),
        in_specs=[pl.BlockSpec((2, BN, HID), lambda i: (0, i, 0))
                  for _ in parts_list] + [
            pl.BlockSpec((BN, HID), lambda i: (i, 0)),
            full(Wo), full(bo), full(g1), full(b1g), full(W1), full(b1),
            full(W2), full(b2), full(g2), full(b2g),
            pl.BlockSpec(_R8.shape, lambda i: (0, 0)),
        ],
        out_specs=pl.BlockSpec((BN, HID), lambda i: (i, 0)),
        out_shape=jax.ShapeDtypeStruct((n, HID), jnp.float32),
    )(*parts_list, h, Wo, bo, g1, b1g, W1, b1, W2, b2, g2, b2g, jnp.asarray(_R8))


def _embed_body(x_ref, w_ref, o_ref):
    o_ref[...] = x_ref[...] @ w_ref[...]


def _embed(x, W):
    n = x.shape[0]
    return pl.pallas_call(
        _embed_body,
        grid=(n // BN,),
        in_specs=[
            pl.BlockSpec((BN, x.shape[1]), lambda i: (i, 0)),
            pl.BlockSpec(W.shape, lambda i: (0, 0)),
        ],
        out_specs=pl.BlockSpec((BN, HID), lambda i: (i, 0)),
        out_shape=jax.ShapeDtypeStruct((n, HID), jnp.float32),
    )(x, W)


_NCORES = 2
_NSUB = 16
_NW = _NCORES * _NSUB


def _chunk(per_w):
    for c in range(128, 7, -8):
        if per_w % c == 0:
            return c
    raise ValueError(per_w)


def _sc_gather(nodeS, nodeD, src, dst):
    e = src.shape[0]
    ws, wd = nodeS.shape[1], nodeD.shape[1]
    per_w = e // _NW
    gch = _chunk(per_w)
    nch = per_w // gch
    mesh = plsc.VectorSubcoreMesh(core_axis_name="c", subcore_axis_name="s")

    @functools.partial(
        pl.kernel, mesh=mesh,
        out_type=[jax.ShapeDtypeStruct((e, ws), jnp.float32),
                  jax.ShapeDtypeStruct((e, wd), jnp.float32)],
        scratch_types=[pltpu.VMEM((gch,), jnp.int32),
                       pltpu.VMEM((gch,), jnp.int32),
                       pltpu.VMEM((gch, ws), jnp.float32),
                       pltpu.VMEM((gch, wd), jnp.float32),
                       pltpu.SemaphoreType.DMA,
                       pltpu.SemaphoreType.DMA],
    )
    def k(ns_hbm, nd_hbm, src_hbm, dst_hbm, gs_hbm, gd_hbm,
          si_v, di_v, rs_v, rd_v, sem1, sem2):
        wid = lax.axis_index("s") * _NCORES + lax.axis_index("c")
        base = wid * per_w

        def body(ci, carry):
            off = base + ci * gch
            pltpu.sync_copy(src_hbm.at[pl.ds(off, gch)], si_v)
            pltpu.sync_copy(dst_hbm.at[pl.ds(off, gch)], di_v)
            cp1 = pltpu.async_copy(ns_hbm.at[si_v], rs_v, sem1)
            cp2 = pltpu.async_copy(nd_hbm.at[di_v], rd_v, sem2)
            cp1.wait()
            cp2.wait()
            pltpu.sync_copy(rs_v, gs_hbm.at[pl.ds(off, gch)])
            pltpu.sync_copy(rd_v, gd_hbm.at[pl.ds(off, gch)])
            return carry

        lax.fori_loop(0, nch, body, 0)

    return k(nodeS, nodeD, src, dst)


def _sc_scatter_add(Y, dst, n):
    e, wy = Y.shape
    per_tile = e // _NSUB
    gch = _chunk(per_tile)
    nch = per_tile // gch
    npad = ((n + 8 * _NSUB - 1) // (8 * _NSUB)) * (8 * _NSUB)
    rows_pt = npad // _NSUB
    mesh = plsc.VectorSubcoreMesh(core_axis_name="c", subcore_axis_name="s")
    zeros_hbm_in = jnp.zeros((npad, HID), jnp.float32)

    @functools.partial(
        pl.kernel, mesh=mesh,
        out_type=jax.ShapeDtypeStruct((_NCORES, npad, HID), jnp.float32),
        scratch_types=[pltpu.VMEM((gch,), jnp.int32),
                       pltpu.VMEM((gch, HID), jnp.float32),
                       pltpu.VMEM_SHARED((npad, HID), jnp.float32),
                       pltpu.SemaphoreType.DMA],
    )
    def k(y_hbm, dst_hbm, zz_hbm, out_hbm, di_v, y_v, acc_sh, sem):
        cid = lax.axis_index("c")
        sid = lax.axis_index("s")

        pltpu.sync_copy(zz_hbm.at[pl.ds(sid * rows_pt, rows_pt)],
                        acc_sh.at[pl.ds(sid * rows_pt, rows_pt)])
        plsc.subcore_barrier()

        base = sid * per_tile
        col = cid * HID

        def body(ci, carry):
            off = base + ci * gch
            pltpu.sync_copy(dst_hbm.at[pl.ds(off, gch)], di_v)
            pltpu.async_copy(y_hbm.at[pl.ds(off, gch), pl.ds(col, HID)],
                             y_v, sem).wait()
            pltpu.sync_copy(y_v, acc_sh.at[di_v], add=True)
            return carry
        lax.fori_loop(0, nch, body, 0)
        plsc.subcore_barrier()

        pltpu.sync_copy(acc_sh.at[pl.ds(sid * rows_pt, rows_pt)],
                        out_hbm.at[cid, pl.ds(sid * rows_pt, rows_pt)])

    return k(Y, dst, zeros_hbm_in)


def _prep_layer(params, p):
    Ms = jnp.concatenate(
        [params['emb_sim_W'] @ p['Wsim'],
         (params['emb_sim_b'] @ p['Wsim'] + p['bsim'])[None, :]], axis=0)
    M5 = Ms.reshape(5, H, DH)
    kr = jnp.asarray(np.kron(np.eye(H, dtype=np.float32), np.ones((DH, 1), np.float32)))
    T = jnp.concatenate([4.0 * Ms[j][:, None] * kr for j in range(5)]
                        + [jnp.zeros((HID, 88), jnp.float32)], axis=1)
    WS = jnp.concatenate([2.0 * p['Wk'], p['Wv'], p['Wk'] @ T], axis=1)
    bS = jnp.concatenate([2.0 * p['bk'], p['bv'], p['bk'] @ T])[None, :]
    WD = jnp.concatenate([2.0 * p['Wq'], p['Wq'] @ T], axis=1)
    bD = jnp.concatenate([2.0 * p['bq'], p['bq'] @ T])[None, :]

    Mp = (params['emb_phi_W'] @ p['Wphi'])[0]
    cp = params['emb_phi_b'] @ p['Wphi'] + p['bphi']
    Psum = Mp.reshape(H, DH).sum(-1)
    Csum = cp.reshape(H, DH).sum(-1)
    G = jnp.einsum('jhd,khd->hjk', M5, M5)
    rows = [4.0 * G[:, 4, 4] + Csum, Psum]
    for i in range(4):
        rows.append(8.0 * G[:, i, 4])
    for i in range(4):
        for j in range(i, 4):
            rows.append((4.0 if i == j else 8.0) * G[:, i, j])
    Wc = jnp.stack(rows, axis=0)
    return WS, bS, WD, bD, Wc


def kernel(x, edge_index, phi, sim, params):
    src = edge_index[0]
    dst = edge_index[1]
    n = x.shape[0]
    e = sim.shape[0]

    layer_prep = [_prep_layer(params, p) for p in params['layers']]
    Wc_all = jnp.concatenate([lp[4] for lp in layer_prep], axis=1)

    h = _embed(x, params['emb_h_W'])
    C_all = _edge_const(sim, phi, Wc_all)

    nsplit = 2
    eh = e // nsplit
    for li, p in enumerate(params['layers']):
        WS, bS, WD, bD, _ = layer_prep[li]
        nodeS, nodeD = _node_tables(h, WS, bS, WD, bD)
        parts_list = []
        for s in range(nsplit):
            sl = slice(s * eh, (s + 1) * eh)
            Gs, Gd = _sc_gather(nodeS, nodeD, src[sl], dst[sl])
            Y = _edge_scores(Gs, Gd, sim[sl], C_all[sl, li * 8:(li + 1) * 8])
            parts_list.append(_sc_scatter_add(Y, dst[sl], n))
        h = _post(parts_list, h, p['Wo'], p['bo'],
                  p['ln1_g'][None, :], p['ln1_b'][None, :],
                  p['W1'], p['b1'][None, :], p['W2'], p['b2'][None, :],
                  p['ln2_g'][None, :], p['ln2_b'][None, :])
    return h


# speedup vs baseline: 23.2160x; 1.1185x over previous
import functools

import jax
import jax.numpy as jnp
import numpy as np
from jax import lax
from jax.experimental import pallas as pl
from jax.experimental.pallas import tpu as pltpu
from jax.experimental.pallas import tpu_sc as plsc

H = 8
DH = 16
HID = 128

_S16 = np.kron(np.eye(H, dtype=np.float32), np.ones((DH, 1), np.float32))
_R8 = np.kron(np.eye(H, dtype=np.float32), np.ones((1, DH), np.float32))

BN = 1000
BE = 1600


def _node_tables_body(h_ref, ws_ref, bs_ref, wd_ref, bd_ref, s_ref, d_ref):
    h = h_ref[...]
    s_ref[...] = h @ ws_ref[...] + bs_ref[...]
    d_ref[...] = h @ wd_ref[...] + bd_ref[...]


def _node_tables(h, WS, bS, WD, bD):
    n = h.shape[0]
    grid = (n // BN,)
    return pl.pallas_call(
        _node_tables_body,
        grid=grid,
        in_specs=[
            pl.BlockSpec((BN, HID), lambda i: (i, 0)),
            pl.BlockSpec(WS.shape, lambda i: (0, 0)),
            pl.BlockSpec(bS.shape, lambda i: (0, 0)),
            pl.BlockSpec(WD.shape, lambda i: (0, 0)),
            pl.BlockSpec(bD.shape, lambda i: (0, 0)),
        ],
        out_specs=[
            pl.BlockSpec((BN, WS.shape[1]), lambda i: (i, 0)),
            pl.BlockSpec((BN, WD.shape[1]), lambda i: (i, 0)),
        ],
        out_shape=[
            jax.ShapeDtypeStruct((n, WS.shape[1]), jnp.float32),
            jax.ShapeDtypeStruct((n, WD.shape[1]), jnp.float32),
        ],
    )(h, WS, bS, WD, bD)


def _edge_const_body(sim_ref, phi_ref, wc_ref, out_ref):
    sim = sim_ref[...]
    wc = wc_ref[...]
    acc = jnp.zeros_like(out_ref[...]) + wc[0][None, :]
    acc = acc + phi_ref[...] * wc[1][None, :]
    feats = []
    for i in range(4):
        feats.append(sim[:, i:i + 1])
    k = 2
    for i in range(4):
        acc = acc + feats[i] * wc[k + i][None, :]
    k = 6
    for i in range(4):
        for j in range(i, 4):
            acc = acc + (feats[i] * feats[j]) * wc[k][None, :]
            k += 1
    out_ref[...] = acc


def _edge_const(sim, phi, Wc):
    e = sim.shape[0]
    return pl.pallas_call(
        _edge_const_body,
        grid=(e // BE,),
        in_specs=[
            pl.BlockSpec((BE, 4), lambda i: (i, 0)),
            pl.BlockSpec((BE, 1), lambda i: (i, 0)),
            pl.BlockSpec(Wc.shape, lambda i: (0, 0)),
        ],
        out_specs=pl.BlockSpec((BE, Wc.shape[1]), lambda i: (i, 0)),
        out_shape=jax.ShapeDtypeStruct((e, Wc.shape[1]), jnp.float32),
    )(sim, phi, Wc)


def _edge_score_body(gs_ref, gd_ref, sim_ref, c_ref, s16_ref, r8_ref, y_ref):
    gs = gs_ref[...]
    gd = gd_ref[...]
    sim = sim_ref[...]
    ks = gs[:, :HID]
    vs = gs[:, HID:2 * HID]
    aK = gs[:, 2 * HID:2 * HID + 48]
    qd = gd[:, :HID]
    aQ = gd[:, HID:HID + 48]
    dots = (ks * qd) @ s16_ref[...]
    a = aK + aQ
    term = a[:, 32:40]
    for j in range(4):
        term = term + sim[:, j:j + 1] * a[:, j * 8:j * 8 + 8]
    s = dots + term + c_ref[...]
    w = jnp.exp(jnp.clip(s, -8.0, 8.0))
    wex = w @ r8_ref[...]
    y_ref[:, :HID] = vs * wex
    y_ref[:, HID:2 * HID] = jnp.pad(w, ((0, 0), (0, HID - 8)))


def _edge_scores(Gs, Gd, sim, C):
    e = sim.shape[0]
    return pl.pallas_call(
        _edge_score_body,
        grid=(e // BE,),
        in_specs=[
            pl.BlockSpec((BE, Gs.shape[1]), lambda i: (i, 0)),
            pl.BlockSpec((BE, Gd.shape[1]), lambda i: (i, 0)),
            pl.BlockSpec((BE, 4), lambda i: (i, 0)),
            pl.BlockSpec((BE, 8), lambda i: (i, 0)),
            pl.BlockSpec(_S16.shape, lambda i: (0, 0)),
            pl.BlockSpec(_R8.shape, lambda i: (0, 0)),
        ],
        out_specs=pl.BlockSpec((BE, 256), lambda i: (i, 0)),
        out_shape=jax.ShapeDtypeStruct((e, 256), jnp.float32),
    )(Gs, Gd, sim, C, jnp.asarray(_S16), jnp.asarray(_R8))


def _ln(x, g, b):
    m = jnp.mean(x, axis=-1, keepdims=True)
    xc = x - m
    v = jnp.mean(xc * xc, axis=-1, keepdims=True)
    return xc * jax.lax.rsqrt(v + 1e-5) * g + b


def _make_post_body(nparts):
    def body(*refs):
        p_refs = refs[:nparts]
        (h_ref, wo_ref, bo_ref, g1_ref, b1g_ref, w1_ref, b1_ref,
         w2_ref, b2_ref, g2_ref, b2g_ref, r8_ref, o_ref) = refs[nparts:]
        wV = p_refs[0][0]
        z = p_refs[0][1][:, :8]
        for pr in p_refs[1:]:
            wV = wV + pr[0]
            z = z + pr[1][:, :8]
        attn = wV / (z @ r8_ref[...] + 1e-6)
        hh = attn @ wo_ref[...] + bo_ref[...]
        r1 = _ln(h_ref[...] + hh, g1_ref[...], b1g_ref[...])
        f = jnp.maximum(r1 @ w1_ref[...] + b1_ref[...], 0.0) @ w2_ref[...] + b2_ref[...]
        o_ref[...] = _ln(r1 + f, g2_ref[...], b2g_ref[...])
    return body


def _post(parts_list, h, Wo, bo, g1, b1g, W1, b1, W2, b2, g2, b2g):
    n = h.shape[0]
    full = lambda a: pl.BlockSpec(a.shape, lambda i: (0,) * a.ndim)
    return pl.pallas_call(
        _make_post_body(len(parts_list)),
        grid=(n // BN,),
        in_specs=[pl.BlockSpec((2, BN, HID), lambda i: (0, i, 0))
                  for _ in parts_list] + [
            pl.BlockSpec((BN, HID), lambda i: (i, 0)),
            full(Wo), full(bo), full(g1), full(b1g), full(W1), full(b1),
            full(W2), full(b2), full(g2), full(b2g),
            pl.BlockSpec(_R8.shape, lambda i: (0, 0)),
        ],
        out_specs=pl.BlockSpec((BN, HID), lambda i: (i, 0)),
        out_shape=jax.ShapeDtypeStruct((n, HID), jnp.float32),
    )(*parts_list, h, Wo, bo, g1, b1g, W1, b1, W2, b2, g2, b2g, jnp.asarray(_R8))


def _embed_body(x_ref, w_ref, o_ref):
    o_ref[...] = x_ref[...] @ w_ref[...]


def _embed(x, W):
    n = x.shape[0]
    return pl.pallas_call(
        _embed_body,
        grid=(n // BN,),
        in_specs=[
            pl.BlockSpec((BN, x.shape[1]), lambda i: (i, 0)),
            pl.BlockSpec(W.shape, lambda i: (0, 0)),
        ],
        out_specs=pl.BlockSpec((BN, HID), lambda i: (i, 0)),
        out_shape=jax.ShapeDtypeStruct((n, HID), jnp.float32),
    )(x, W)


_NCORES = 2
_NSUB = 16
_NW = _NCORES * _NSUB


_GCH = 128


def _sc_gather(nodeS, nodeD, src, dst):
    e = src.shape[0]
    ws, wd = nodeS.shape[1], nodeD.shape[1]
    nchunks = e // _GCH
    iters = (nchunks + _NW - 1) // _NW
    mesh = plsc.VectorSubcoreMesh(core_axis_name="c", subcore_axis_name="s")

    @functools.partial(
        pl.kernel, mesh=mesh,
        out_type=[jax.ShapeDtypeStruct((e, ws), jnp.float32),
                  jax.ShapeDtypeStruct((e, wd), jnp.float32)],
        scratch_types=[pltpu.VMEM((_GCH,), jnp.int32),
                       pltpu.VMEM((_GCH,), jnp.int32),
                       pltpu.VMEM((_GCH, ws), jnp.float32),
                       pltpu.VMEM((_GCH, wd), jnp.float32),
                       pltpu.SemaphoreType.DMA,
                       pltpu.SemaphoreType.DMA],
    )
    def k(ns_hbm, nd_hbm, src_hbm, dst_hbm, gs_hbm, gd_hbm,
          si_v, di_v, rs_v, rd_v, sem1, sem2):
        wid = lax.axis_index("s") * _NCORES + lax.axis_index("c")

        def body(ci, carry):
            cid = wid + ci * _NW

            @pl.when(cid < nchunks)
            def _():
                off = cid * _GCH
                pltpu.sync_copy(src_hbm.at[pl.ds(off, _GCH)], si_v)
                pltpu.sync_copy(dst_hbm.at[pl.ds(off, _GCH)], di_v)
                cp1 = pltpu.async_copy(ns_hbm.at[si_v], rs_v, sem1)
                cp2 = pltpu.async_copy(nd_hbm.at[di_v], rd_v, sem2)
                cp1.wait()
                cp2.wait()
                pltpu.sync_copy(rs_v, gs_hbm.at[pl.ds(off, _GCH)])
                pltpu.sync_copy(rd_v, gd_hbm.at[pl.ds(off, _GCH)])
            return carry

        lax.fori_loop(0, iters, body, 0)

    return k(nodeS, nodeD, src, dst)


def _sc_scatter_add(Y, dst, n):
    e, wy = Y.shape
    nchunks = e // _GCH
    iters = (nchunks + _NSUB - 1) // _NSUB
    npad = ((n + 8 * _NSUB - 1) // (8 * _NSUB)) * (8 * _NSUB)
    rows_pt = npad // _NSUB
    mesh = plsc.VectorSubcoreMesh(core_axis_name="c", subcore_axis_name="s")
    zeros_hbm_in = jnp.zeros((npad, HID), jnp.float32)

    @functools.partial(
        pl.kernel, mesh=mesh,
        out_type=jax.ShapeDtypeStruct((_NCORES, npad, HID), jnp.float32),
        scratch_types=[pltpu.VMEM((_GCH,), jnp.int32),
                       pltpu.VMEM((_GCH, HID), jnp.float32),
                       pltpu.VMEM_SHARED((npad, HID), jnp.float32),
                       pltpu.SemaphoreType.DMA],
    )
    def k(y_hbm, dst_hbm, zz_hbm, out_hbm, di_v, y_v, acc_sh, sem):
        cid = lax.axis_index("c")
        sid = lax.axis_index("s")

        pltpu.sync_copy(zz_hbm.at[pl.ds(sid * rows_pt, rows_pt)],
                        acc_sh.at[pl.ds(sid * rows_pt, rows_pt)])
        plsc.subcore_barrier()

        col = cid * HID

        def body(ci, carry):
            ch = sid + ci * _NSUB

            @pl.when(ch < nchunks)
            def _():
                off = ch * _GCH
                pltpu.sync_copy(dst_hbm.at[pl.ds(off, _GCH)], di_v)
                pltpu.async_copy(y_hbm.at[pl.ds(off, _GCH), pl.ds(col, HID)],
                                 y_v, sem).wait()
                pltpu.sync_copy(y_v, acc_sh.at[di_v], add=True)
            return carry
        lax.fori_loop(0, iters, body, 0)
        plsc.subcore_barrier()

        pltpu.sync_copy(acc_sh.at[pl.ds(sid * rows_pt, rows_pt)],
                        out_hbm.at[cid, pl.ds(sid * rows_pt, rows_pt)])

    return k(Y, dst, zeros_hbm_in)


def _prep_layer(params, p):
    Ms = jnp.concatenate(
        [params['emb_sim_W'] @ p['Wsim'],
         (params['emb_sim_b'] @ p['Wsim'] + p['bsim'])[None, :]], axis=0)
    M5 = Ms.reshape(5, H, DH)
    kr = jnp.asarray(np.kron(np.eye(H, dtype=np.float32), np.ones((DH, 1), np.float32)))
    T = jnp.concatenate([4.0 * Ms[j][:, None] * kr for j in range(5)]
                        + [jnp.zeros((HID, 88), jnp.float32)], axis=1)
    WS = jnp.concatenate([2.0 * p['Wk'], p['Wv'], p['Wk'] @ T], axis=1)
    bS = jnp.concatenate([2.0 * p['bk'], p['bv'], p['bk'] @ T])[None, :]
    WD = jnp.concatenate([2.0 * p['Wq'], p['Wq'] @ T], axis=1)
    bD = jnp.concatenate([2.0 * p['bq'], p['bq'] @ T])[None, :]

    Mp = (params['emb_phi_W'] @ p['Wphi'])[0]
    cp = params['emb_phi_b'] @ p['Wphi'] + p['bphi']
    Psum = Mp.reshape(H, DH).sum(-1)
    Csum = cp.reshape(H, DH).sum(-1)
    G = jnp.einsum('jhd,khd->hjk', M5, M5)
    rows = [4.0 * G[:, 4, 4] + Csum, Psum]
    for i in range(4):
        rows.append(8.0 * G[:, i, 4])
    for i in range(4):
        for j in range(i, 4):
            rows.append((4.0 if i == j else 8.0) * G[:, i, j])
    Wc = jnp.stack(rows, axis=0)
    return WS, bS, WD, bD, Wc


def kernel(x, edge_index, phi, sim, params):
    src = edge_index[0]
    dst = edge_index[1]
    n = x.shape[0]
    e = sim.shape[0]

    layer_prep = [_prep_layer(params, p) for p in params['layers']]
    Wc_all = jnp.concatenate([lp[4] for lp in layer_prep], axis=1)

    h = _embed(x, params['emb_h_W'])
    C_all = _edge_const(sim, phi, Wc_all)

    nsplit = 2
    eh = e // nsplit
    for li, p in enumerate(params['layers']):
        WS, bS, WD, bD, _ = layer_prep[li]
        nodeS, nodeD = _node_tables(h, WS, bS, WD, bD)
        parts_list = []
        for s in range(nsplit):
            sl = slice(s * eh, (s + 1) * eh)
            Gs, Gd = _sc_gather(nodeS, nodeD, src[sl], dst[sl])
            Y = _edge_scores(Gs, Gd, sim[sl], C_all[sl, li * 8:(li + 1) * 8])
            parts_list.append(_sc_scatter_add(Y, dst[sl], n))
        h = _post(parts_list, h, p['Wo'], p['bo'],
                  p['ln1_g'][None, :], p['ln1_b'][None, :],
                  p['W1'], p['b1'][None, :], p['W2'], p['b2'][None, :],
                  p['ln2_g'][None, :], p['ln2_b'][None, :])
    return h


# speedup vs baseline: 36.1801x; 1.5584x over previous
import functools

import jax
import jax.numpy as jnp
import numpy as np
from jax import lax
from jax.experimental import pallas as pl
from jax.experimental.pallas import tpu as pltpu
from jax.experimental.pallas import tpu_sc as plsc

H = 8
DH = 16
HID = 128

_S16 = np.kron(np.eye(H, dtype=np.float32), np.ones((DH, 1), np.float32))
_R8 = np.kron(np.eye(H, dtype=np.float32), np.ones((1, DH), np.float32))

BN = 1000
BE = 1600


def _node_tables_body(h_ref, ws_ref, bs_ref, wd_ref, bd_ref, s_ref, d_ref):
    h = h_ref[...]
    s_ref[...] = h @ ws_ref[...] + bs_ref[...]
    d_ref[...] = h @ wd_ref[...] + bd_ref[...]


def _node_tables(h, WS, bS, WD, bD):
    n = h.shape[0]
    grid = (n // BN,)
    return pl.pallas_call(
        _node_tables_body,
        grid=grid,
        in_specs=[
            pl.BlockSpec((BN, HID), lambda i: (i, 0)),
            pl.BlockSpec(WS.shape, lambda i: (0, 0)),
            pl.BlockSpec(bS.shape, lambda i: (0, 0)),
            pl.BlockSpec(WD.shape, lambda i: (0, 0)),
            pl.BlockSpec(bD.shape, lambda i: (0, 0)),
        ],
        out_specs=[
            pl.BlockSpec((BN, WS.shape[1]), lambda i: (i, 0)),
            pl.BlockSpec((BN, WD.shape[1]), lambda i: (i, 0)),
        ],
        out_shape=[
            jax.ShapeDtypeStruct((n, WS.shape[1]), jnp.float32),
            jax.ShapeDtypeStruct((n, WD.shape[1]), jnp.float32),
        ],
    )(h, WS, bS, WD, bD)


def _edge_score_body(gs_ref, gd_ref, sim_ref, phi_ref, ms_ref, c2_ref,
                     pc_ref, s16_ref, r8_ref, y_ref):
    gs = gs_ref[...]
    ks = gs[:, :HID]
    vs = gs[:, HID:2 * HID]
    qd = gd_ref[...]
    S2 = sim_ref[...] @ ms_ref[...] + c2_ref[...]
    a = ks + S2
    b = qd + S2
    dots = (a * b) @ s16_ref[...]
    pc = pc_ref[...]
    s = dots + phi_ref[...] * pc[0][None, :] + pc[1][None, :]
    w = jnp.exp(jnp.clip(s, -8.0, 8.0))
    wex = w @ r8_ref[...]
    y_ref[:, :HID] = vs * wex
    y_ref[:, HID:2 * HID] = jnp.pad(w, ((0, 0), (0, HID - 8)))


def _edge_scores(Gs, Gd, sim, phi, Ms2, c2, pc):
    e = sim.shape[0]
    return pl.pallas_call(
        _edge_score_body,
        grid=(e // BE,),
        in_specs=[
            pl.BlockSpec((BE, Gs.shape[1]), lambda i: (i, 0)),
            pl.BlockSpec((BE, Gd.shape[1]), lambda i: (i, 0)),
            pl.BlockSpec((BE, 4), lambda i: (i, 0)),
            pl.BlockSpec((BE, 1), lambda i: (i, 0)),
            pl.BlockSpec(Ms2.shape, lambda i: (0, 0)),
            pl.BlockSpec(c2.shape, lambda i: (0, 0)),
            pl.BlockSpec(pc.shape, lambda i: (0, 0)),
            pl.BlockSpec(_S16.shape, lambda i: (0, 0)),
            pl.BlockSpec(_R8.shape, lambda i: (0, 0)),
        ],
        out_specs=pl.BlockSpec((BE, 256), lambda i: (i, 0)),
        out_shape=jax.ShapeDtypeStruct((e, 256), jnp.float32),
    )(Gs, Gd, sim, phi, Ms2, c2, pc, jnp.asarray(_S16), jnp.asarray(_R8))


def _ln(x, g, b):
    m = jnp.mean(x, axis=-1, keepdims=True)
    xc = x - m
    v = jnp.mean(xc * xc, axis=-1, keepdims=True)
    return xc * jax.lax.rsqrt(v + 1e-5) * g + b


def _make_post_body(nparts):
    def body(*refs):
        p_refs = refs[:nparts]
        (h_ref, wo_ref, bo_ref, g1_ref, b1g_ref, w1_ref, b1_ref,
         w2_ref, b2_ref, g2_ref, b2g_ref, r8_ref, o_ref) = refs[nparts:]
        wV = p_refs[0][0]
        z = p_refs[0][1][:, :8]
        for pr in p_refs[1:]:
            wV = wV + pr[0]
            z = z + pr[1][:, :8]
        attn = wV / (z @ r8_ref[...] + 1e-6)
        hh = attn @ wo_ref[...] + bo_ref[...]
        r1 = _ln(h_ref[...] + hh, g1_ref[...], b1g_ref[...])
        f = jnp.maximum(r1 @ w1_ref[...] + b1_ref[...], 0.0) @ w2_ref[...] + b2_ref[...]
        o_ref[...] = _ln(r1 + f, g2_ref[...], b2g_ref[...])
    return body


def _post(parts_list, h, Wo, bo, g1, b1g, W1, b1, W2, b2, g2, b2g):
    n = h.shape[0]
    full = lambda a: pl.BlockSpec(a.shape, lambda i: (0,) * a.ndim)
    return pl.pallas_call(
        _make_post_body(len(parts_list)),
        grid=(n // BN,),
        in_specs=[pl.BlockSpec((2, BN, p.shape[2]), lambda i: (0, i, 0))
                  for p in parts_list] + [
            pl.BlockSpec((BN, HID), lambda i: (i, 0)),
            full(Wo), full(bo), full(g1), full(b1g), full(W1), full(b1),
            full(W2), full(b2), full(g2), full(b2g),
            pl.BlockSpec(_R8.shape, lambda i: (0, 0)),
        ],
        out_specs=pl.BlockSpec((BN, HID), lambda i: (i, 0)),
        out_shape=jax.ShapeDtypeStruct((n, HID), jnp.float32),
    )(*parts_list, h, Wo, bo, g1, b1g, W1, b1, W2, b2, g2, b2g, jnp.asarray(_R8))


def _embed_body(x_ref, w_ref, o_ref):
    o_ref[...] = x_ref[...] @ w_ref[...]


def _embed(x, W):
    n = x.shape[0]
    return pl.pallas_call(
        _embed_body,
        grid=(n // BN,),
        in_specs=[
            pl.BlockSpec((BN, x.shape[1]), lambda i: (i, 0)),
            pl.BlockSpec(W.shape, lambda i: (0, 0)),
        ],
        out_specs=pl.BlockSpec((BN, HID), lambda i: (i, 0)),
        out_shape=jax.ShapeDtypeStruct((n, HID), jnp.float32),
    )(x, W)


_NCORES = 2
_NSUB = 16
_NW = _NCORES * _NSUB


_GCH = 128


def _sc_gather(nodeS, nodeD, src, dst):
    e = src.shape[0]
    ws, wd = nodeS.shape[1], nodeD.shape[1]
    nchunks = e // _GCH
    iters = (nchunks + _NW - 1) // _NW
    mesh = plsc.VectorSubcoreMesh(core_axis_name="c", subcore_axis_name="s")

    @functools.partial(
        pl.kernel, mesh=mesh,
        out_type=[jax.ShapeDtypeStruct((e, ws), jnp.float32),
                  jax.ShapeDtypeStruct((e, wd), jnp.float32)],
        scratch_types=[pltpu.VMEM((_GCH,), jnp.int32),
                       pltpu.VMEM((_GCH,), jnp.int32),
                       pltpu.VMEM((_GCH, ws), jnp.float32),
                       pltpu.VMEM((_GCH, wd), jnp.float32),
                       pltpu.SemaphoreType.DMA,
                       pltpu.SemaphoreType.DMA],
    )
    def k(ns_hbm, nd_hbm, src_hbm, dst_hbm, gs_hbm, gd_hbm,
          si_v, di_v, rs_v, rd_v, sem1, sem2):
        wid = lax.axis_index("s") * _NCORES + lax.axis_index("c")

        def body(ci, carry):
            cid = wid + ci * _NW

            @pl.when(cid < nchunks)
            def _():
                off = cid * _GCH
                pltpu.sync_copy(src_hbm.at[pl.ds(off, _GCH)], si_v)
                pltpu.sync_copy(dst_hbm.at[pl.ds(off, _GCH)], di_v)
                cp1 = pltpu.async_copy(ns_hbm.at[si_v], rs_v, sem1)
                cp2 = pltpu.async_copy(nd_hbm.at[di_v], rd_v, sem2)
                cp1.wait()
                cp2.wait()
                pltpu.sync_copy(rs_v, gs_hbm.at[pl.ds(off, _GCH)])
                pltpu.sync_copy(rd_v, gd_hbm.at[pl.ds(off, _GCH)])
            return carry

        lax.fori_loop(0, iters, body, 0)

    return k(nodeS, nodeD, src, dst)


def _sc_scatter_add(Y, dst, n):
    e, wy = Y.shape
    nchunks = e // _GCH
    iters = (nchunks + _NSUB - 1) // _NSUB
    npad = ((n + 8 * _NSUB - 1) // (8 * _NSUB)) * (8 * _NSUB)
    rows_pt = npad // _NSUB
    mesh = plsc.VectorSubcoreMesh(core_axis_name="c", subcore_axis_name="s")
    zeros_hbm_in = jnp.zeros((npad, HID), jnp.float32)

    @functools.partial(
        pl.kernel, mesh=mesh,
        out_type=jax.ShapeDtypeStruct((_NCORES, npad, HID), jnp.float32),
        scratch_types=[pltpu.VMEM((_GCH,), jnp.int32),
                       pltpu.VMEM((_GCH, HID), jnp.float32),
                       pltpu.VMEM_SHARED((npad, HID), jnp.float32),
                       pltpu.SemaphoreType.DMA],
    )
    def k(y_hbm, dst_hbm, zz_hbm, out_hbm, di_v, y_v, acc_sh, sem):
        cid = lax.axis_index("c")
        sid = lax.axis_index("s")

        pltpu.sync_copy(zz_hbm.at[pl.ds(sid * rows_pt, rows_pt)],
                        acc_sh.at[pl.ds(sid * rows_pt, rows_pt)])
        plsc.subcore_barrier()

        col = cid * HID

        def body(ci, carry):
            ch = sid + ci * _NSUB

            @pl.when(ch < nchunks)
            def _():
                off = ch * _GCH
                pltpu.sync_copy(dst_hbm.at[pl.ds(off, _GCH)], di_v)
                pltpu.async_copy(y_hbm.at[pl.ds(off, _GCH), pl.ds(col, HID)],
                                 y_v, sem).wait()
                pltpu.sync_copy(y_v, acc_sh.at[di_v], add=True)
            return carry
        lax.fori_loop(0, iters, body, 0)
        plsc.subcore_barrier()

        pltpu.sync_copy(acc_sh.at[pl.ds(sid * rows_pt, rows_pt)],
                        out_hbm.at[cid, pl.ds(sid * rows_pt, rows_pt)])

    return k(Y, dst, zeros_hbm_in)


def _prep_layer(params, p):
    WS = jnp.concatenate([2.0 * p['Wk'], p['Wv']], axis=1)
    bS = jnp.concatenate([2.0 * p['bk'], p['bv']])[None, :]
    WD = 2.0 * p['Wq']
    bD = (2.0 * p['bq'])[None, :]

    Ms2 = 2.0 * (params['emb_sim_W'] @ p['Wsim'])
    c2 = 2.0 * (params['emb_sim_b'] @ p['Wsim'] + p['bsim'])[None, :]

    Mp = (params['emb_phi_W'] @ p['Wphi'])[0]
    cp = params['emb_phi_b'] @ p['Wphi'] + p['bphi']
    Psum = Mp.reshape(H, DH).sum(-1)
    Csum = cp.reshape(H, DH).sum(-1)
    pc = jnp.stack([Psum, Csum], axis=0)
    return WS, bS, WD, bD, Ms2, c2, pc


def kernel(x, edge_index, phi, sim, params):
    src = edge_index[0]
    dst = edge_index[1]
    n = x.shape[0]
    e = sim.shape[0]

    layer_prep = [_prep_layer(params, p) for p in params['layers']]

    h = _embed(x, params['emb_h_W'])

    nsplit = 2
    eh = e // nsplit
    for li, p in enumerate(params['layers']):
        WS, bS, WD, bD, Ms2, c2, pc = layer_prep[li]
        nodeS, nodeD = _node_tables(h, WS, bS, WD, bD)
        parts_list = []
        for s in range(nsplit):
            sl = slice(s * eh, (s + 1) * eh)
            Gs, Gd = _sc_gather(nodeS, nodeD, src[sl], dst[sl])
            Y = _edge_scores(Gs, Gd, sim[sl], phi[sl], Ms2, c2, pc)
            parts_list.append(_sc_scatter_add(Y, dst[sl], n))
        h = _post(parts_list, h, p['Wo'], p['bo'],
                  p['ln1_g'][None, :], p['ln1_b'][None, :],
                  p['W1'], p['b1'][None, :], p['W2'], p['b2'][None, :],
                  p['ln2_g'][None, :], p['ln2_b'][None, :])
    return h
